# baseline (speedup 1.0000x reference)
import functools

import jax
import jax.numpy as jnp
from jax import lax
from jax.experimental import pallas as pl
from jax.experimental.pallas import tpu as pltpu
from jax.experimental.pallas import tpu_sc as plsc

_NUM_WORKERS = 32
_CHUNK = 200


def kernel(V_set, E_set, node_ids):
    V = V_set[0]
    E = E_set[0]
    idx = node_ids[0].astype(jnp.int32)
    M, De = E.shape
    D = V.shape[1]
    b_per_w = M // _NUM_WORKERS
    n_chunks = b_per_w // _CHUNK
    n_pairs = n_chunks // 2

    mesh = plsc.VectorSubcoreMesh(core_axis_name="c", subcore_axis_name="s")

    @functools.partial(
        pl.kernel,
        mesh=mesh,
        out_type=jax.ShapeDtypeStruct((M, De + D), jnp.float32),
        scratch_types=[
            pltpu.VMEM((b_per_w,), jnp.int32),
            pltpu.VMEM((_CHUNK, D), jnp.float32),
            pltpu.VMEM((_CHUNK, D), jnp.float32),
            pltpu.SemaphoreType.DMA,
            pltpu.SemaphoreType.DMA,
            pltpu.SemaphoreType.DMA,
            pltpu.SemaphoreType.DMA,
            pltpu.SemaphoreType.DMA,
        ],
        compiler_params=pltpu.CompilerParams(use_tc_tiling_on_sc=False),
    )
    def _k(v_hbm, e_hbm, idx_hbm, out_hbm, idx_all, rows0, rows1,
           g0, g1, o0, o1, se):
        wid = lax.axis_index("s") * 2 + lax.axis_index("c")
        base = wid * b_per_w
        rows = (rows0, rows1)
        sg = (g0, g1)
        so = (o0, o1)

        def gather_desc(c, b):
            return pltpu.make_async_copy(
                v_hbm.at[idx_all.at[pl.ds(c * _CHUNK, _CHUNK)]], rows[b], sg[b])

        def out_desc(c, b):
            return pltpu.make_async_copy(
                rows[b],
                out_hbm.at[pl.ds(base + c * _CHUNK, _CHUNK), pl.ds(De, D)],
                so[b])

        pltpu.sync_copy(idx_hbm.at[pl.ds(base, b_per_w)], idx_all)
        e_desc = pltpu.make_async_copy(
            e_hbm.at[pl.ds(base, b_per_w)],
            out_hbm.at[pl.ds(base, b_per_w), pl.ds(0, De)], se)
        e_desc.start()
        gather_desc(0, 0).start()

        def body(c2, carry):
            for b in (0, 1):
                c = 2 * c2 + b
                gather_desc(c, b).wait()
                out_desc(c, b).start()
                nb = 1 - b
                if b == 0:
                    @pl.when(c2 >= 1)
                    def _():
                        out_desc(c - 1, nb).wait()
                    gather_desc(c + 1, nb).start()
                else:
                    @pl.when(c2 < n_pairs - 1)
                    def _():
                        out_desc(c - 1, nb).wait()
                        gather_desc(c + 1, nb).start()
            return carry

        lax.fori_loop(0, n_pairs, body, 0)

        out_desc(n_chunks - 2, 0).wait()
        out_desc(n_chunks - 1, 1).wait()
        e_desc.wait()

    out = _k(V, E, idx)
    return out[jnp.newaxis]


# speedup vs baseline: 1.0021x; 1.0021x over previous
import functools

import jax
import jax.numpy as jnp
from jax import lax
from jax.experimental import pallas as pl
from jax.experimental.pallas import tpu as pltpu
from jax.experimental.pallas import tpu_sc as plsc

_NUM_WORKERS = 32
_CHUNK = 200


def kernel(V_set, E_set, node_ids):
    V = V_set[0]
    E = E_set[0]
    idx = node_ids[0].astype(jnp.int32)
    M, De = E.shape
    D = V.shape[1]
    b_per_w = M // _NUM_WORKERS
    n_chunks = b_per_w // _CHUNK
    n_pairs = n_chunks // 2

    mesh = plsc.VectorSubcoreMesh(core_axis_name="c", subcore_axis_name="s")

    @functools.partial(
        pl.kernel,
        mesh=mesh,
        out_type=jax.ShapeDtypeStruct((M, De + D), jnp.float32),
        scratch_types=[
            pltpu.VMEM((_CHUNK,), jnp.int32),
            pltpu.VMEM((_CHUNK,), jnp.int32),
            pltpu.VMEM((_CHUNK, D), jnp.float32),
            pltpu.VMEM((_CHUNK, D), jnp.float32),
            pltpu.SemaphoreType.DMA,
            pltpu.SemaphoreType.DMA,
            pltpu.SemaphoreType.DMA,
            pltpu.SemaphoreType.DMA,
            pltpu.SemaphoreType.DMA,
        ],
        compiler_params=pltpu.CompilerParams(use_tc_tiling_on_sc=False),
    )
    def _k(v_hbm, e_hbm, idx_hbm, out_hbm, idx0, idx1, rows0, rows1,
           g0, g1, o0, o1, se):
        wid = lax.axis_index("s") * 2 + lax.axis_index("c")
        base = wid * b_per_w
        idxs = (idx0, idx1)
        rows = (rows0, rows1)
        sg = (g0, g1)
        so = (o0, o1)

        def gather_start(c, b):
            pltpu.sync_copy(idx_hbm.at[pl.ds(base + c * _CHUNK, _CHUNK)],
                            idxs[b])
            pltpu.make_async_copy(v_hbm.at[idxs[b]], rows[b], sg[b]).start()

        def gather_wait(b):
            pltpu.make_async_copy(v_hbm.at[idxs[b]], rows[b], sg[b]).wait()

        def out_desc(c, b):
            return pltpu.make_async_copy(
                rows[b],
                out_hbm.at[pl.ds(base + c * _CHUNK, _CHUNK), pl.ds(De, D)],
                so[b])

        e_desc = pltpu.make_async_copy(
            e_hbm.at[pl.ds(base, b_per_w)],
            out_hbm.at[pl.ds(base, b_per_w), pl.ds(0, De)], se)
        e_desc.start()
        gather_start(0, 0)

        def body(c2, carry):
            for b in (0, 1):
                c = 2 * c2 + b
                gather_wait(b)
                out_desc(c, b).start()
                nb = 1 - b
                if b == 0:
                    @pl.when(c2 >= 1)
                    def _():
                        out_desc(c - 1, nb).wait()
                    gather_start(c + 1, nb)
                else:
                    @pl.when(c2 < n_pairs - 1)
                    def _():
                        out_desc(c - 1, nb).wait()
                        gather_start(c + 1, nb)
            return carry

        lax.fori_loop(0, n_pairs, body, 0)

        out_desc(n_chunks - 2, 0).wait()
        out_desc(n_chunks - 1, 1).wait()
        e_desc.wait()

    out = _k(V, E, idx)
    return out[jnp.newaxis]


# speedup vs baseline: 2.3543x; 2.3495x over previous
import functools

import jax
import jax.numpy as jnp
from jax import lax
from jax.experimental import pallas as pl
from jax.experimental.pallas import tpu as pltpu
from jax.experimental.pallas import tpu_sc as plsc

_NUM_WORKERS = 32
_CHUNK = 200


def kernel(V_set, E_set, node_ids):
    V = V_set[0]
    E = E_set[0]
    idx = node_ids[0].astype(jnp.int32)
    M, De = E.shape
    D = V.shape[1]
    b_per_w = M // _NUM_WORKERS
    n_chunks = b_per_w // _CHUNK
    n_pairs = n_chunks // 2

    mesh = plsc.VectorSubcoreMesh(core_axis_name="c", subcore_axis_name="s")

    @functools.partial(
        pl.kernel,
        mesh=mesh,
        out_type=jax.ShapeDtypeStruct((M, De + D), jnp.float32),
        scratch_types=[
            pltpu.VMEM((_CHUNK,), jnp.int32),
            pltpu.VMEM((_CHUNK,), jnp.int32),
            pltpu.VMEM((_CHUNK, D), jnp.float32),
            pltpu.VMEM((_CHUNK, D), jnp.float32),
            pltpu.VMEM((_CHUNK, De), jnp.float32),
            pltpu.VMEM((_CHUNK, De), jnp.float32),
            pltpu.SemaphoreType.DMA,
            pltpu.SemaphoreType.DMA,
            pltpu.SemaphoreType.DMA,
            pltpu.SemaphoreType.DMA,
            pltpu.SemaphoreType.DMA,
            pltpu.SemaphoreType.DMA,
        ],
        compiler_params=pltpu.CompilerParams(use_tc_tiling_on_sc=False),
    )
    def _k(v_hbm, e_hbm, idx_hbm, out_hbm, idx0, idx1, rows0, rows1,
           ev0, ev1, g0, g1, o0, o1, se0, se1):
        wid = lax.axis_index("s") * 2 + lax.axis_index("c")
        base = wid * b_per_w
        idxs = (idx0, idx1)
        rows = (rows0, rows1)
        evs = (ev0, ev1)
        sg = (g0, g1)
        so = (o0, o1)
        se = (se0, se1)

        def gather_start(c, b):
            pltpu.sync_copy(idx_hbm.at[pl.ds(base + c * _CHUNK, _CHUNK)],
                            idxs[b])
            pltpu.make_async_copy(v_hbm.at[idxs[b]], rows[b], sg[b]).start()
            pltpu.make_async_copy(
                e_hbm.at[pl.ds(base + c * _CHUNK, _CHUNK)], evs[b],
                se[b]).start()

        def gather_wait(c, b):
            pltpu.make_async_copy(v_hbm.at[idxs[b]], rows[b], sg[b]).wait()
            pltpu.make_async_copy(
                e_hbm.at[pl.ds(base + c * _CHUNK, _CHUNK)], evs[b],
                se[b]).wait()

        def out_start(c, b):
            pltpu.make_async_copy(
                rows[b],
                out_hbm.at[pl.ds(base + c * _CHUNK, _CHUNK), pl.ds(De, D)],
                so[b]).start()
            pltpu.make_async_copy(
                evs[b],
                out_hbm.at[pl.ds(base + c * _CHUNK, _CHUNK), pl.ds(0, De)],
                so[b]).start()

        def out_wait(c, b):
            pltpu.make_async_copy(
                rows[b],
                out_hbm.at[pl.ds(base + c * _CHUNK, _CHUNK), pl.ds(De, D)],
                so[b]).wait()
            pltpu.make_async_copy(
                evs[b],
                out_hbm.at[pl.ds(base + c * _CHUNK, _CHUNK), pl.ds(0, De)],
                so[b]).wait()

        gather_start(0, 0)

        def body(c2, carry):
            for b in (0, 1):
                c = 2 * c2 + b
                gather_wait(c, b)
                out_start(c, b)
                nb = 1 - b
                if b == 0:
                    @pl.when(c2 >= 1)
                    def _():
                        out_wait(c - 1, nb)
                    gather_start(c + 1, nb)
                else:
                    @pl.when(c2 < n_pairs - 1)
                    def _():
                        out_wait(c - 1, nb)
                        gather_start(c + 1, nb)
            return carry

        lax.fori_loop(0, n_pairs, body, 0)

        out_wait(n_chunks - 2, 0)
        out_wait(n_chunks - 1, 1)

    out = _k(V, E, idx)
    return out[jnp.newaxis]
